# baseline (speedup 1.0000x reference)
import functools

import jax
import jax.numpy as jnp
from jax.experimental import pallas as pl
from jax.experimental.pallas import tpu as pltpu

VSX, VSY = 0.1, 0.1
XMIN, YMIN, ZMIN, XMAX, YMAX, ZMAX = 0.0, -25.6, -3.0, 51.2, 25.6, 3.0
GX = 512
GY = 512
SIG2 = (VSX * 0.5) ** 2

_ROWS_PER_BLOCK = 256


def _gauss_kernel(sx_ref, sy_ref, cnt_ref, pm_ref, out_ref):
    i = pl.program_id(0)
    rows = jax.lax.broadcasted_iota(jnp.int32, out_ref.shape, 0) + i * _ROWS_PER_BLOCK
    gx = jax.lax.rem(rows, GX)
    gy = jax.lax.broadcasted_iota(jnp.int32, out_ref.shape, 1)
    xc = XMIN + (gx.astype(jnp.float32) + 0.5) * VSX
    yc = YMIN + (gy.astype(jnp.float32) + 0.5) * VSY
    cnt = cnt_ref[...]
    occ = (cnt > 0.0).astype(jnp.float32)
    denom = jnp.maximum(cnt, 1.0)
    mx = sx_ref[...] / denom
    my = sy_ref[...] / denom
    d2 = (mx - xc) ** 2 + (my - yc) ** 2
    out_ref[...] = jnp.exp(-d2 / (2.0 * SIG2)) * jnp.tanh(pm_ref[...]) * occ


def _segment_embed(points, W, b):
    B, n, C = points.shape
    xyz = points[..., :3]
    cx = jnp.floor((xyz[..., 0] - XMIN) / VSX).astype(jnp.int32)
    cy = jnp.floor((xyz[..., 1] - YMIN) / VSY).astype(jnp.int32)
    valid = (cx >= 0) & (cx < GX) & (cy >= 0) & (cy < GY) & (xyz[..., 2] >= ZMIN) & (xyz[..., 2] < ZMAX)
    bidx = jnp.arange(B, dtype=jnp.int32)[:, None]
    seg = jnp.where(valid, bidx * (GX * GY) + cx * GY + cy, B * GX * GY)
    seg = seg.reshape(-1)
    S = B * GX * GY + 1
    flat = xyz.reshape(B * n, 3)
    ones = jnp.ones((B * n,), jnp.float32)
    sx = jax.ops.segment_sum(flat[:, 0], seg, num_segments=S)
    sy = jax.ops.segment_sum(flat[:, 1], seg, num_segments=S)
    cnt = jax.ops.segment_sum(ones, seg, num_segments=S)
    h = jax.nn.relu(points.reshape(B * n, C) @ W + b)
    pmax = jax.ops.segment_max(h, seg, num_segments=S)
    pmax = jnp.where((cnt > 0)[:, None], pmax, 0.0)
    pm = jnp.mean(pmax, -1)
    return (sx[: S - 1].reshape(B, GX, GY), sy[: S - 1].reshape(B, GX, GY),
            cnt[: S - 1].reshape(B, GX, GY), pm[: S - 1].reshape(B, GX, GY))


def kernel(pc0, pc1, radar_pc0, radar_pc1, pose0, pose1, W_lidar, b_lidar, W_radar, b_radar):
    B = pc0.shape[0]
    pose_0to1 = jnp.linalg.inv(pose1) @ pose0
    R = pose_0to1[:, :3, :3]
    t = pose_0to1[:, :3, 3]
    tpc0 = jnp.einsum("bnc,bdc->bnd", pc0, R) + t[:, None, :]
    rxyz = jnp.einsum("bnc,bdc->bnd", radar_pc0[..., :3], R) + t[:, None, :]
    tr0 = jnp.concatenate([rxyz, radar_pc0[..., 3:]], -1)

    parts = [
        _segment_embed(tpc0, W_lidar, b_lidar),
        _segment_embed(tr0, W_radar, b_radar),
        _segment_embed(pc1, W_lidar, b_lidar),
        _segment_embed(radar_pc1, W_radar, b_radar),
    ]
    sx = jnp.stack([p[0] for p in parts], 0).reshape(4 * B * GX, GY)
    sy = jnp.stack([p[1] for p in parts], 0).reshape(4 * B * GX, GY)
    cnt = jnp.stack([p[2] for p in parts], 0).reshape(4 * B * GX, GY)
    pm = jnp.stack([p[3] for p in parts], 0).reshape(4 * B * GX, GY)

    nrows = 4 * B * GX
    grid = (nrows // _ROWS_PER_BLOCK,)
    spec = pl.BlockSpec((_ROWS_PER_BLOCK, GY), lambda i: (i, 0))
    out = pl.pallas_call(
        _gauss_kernel,
        grid=grid,
        in_specs=[spec, spec, spec, spec],
        out_specs=spec,
        out_shape=jax.ShapeDtypeStruct((nrows, GY), jnp.float32),
    )(sx, sy, cnt, pm)
    return out.reshape(4, B, GX, GY)


# speedup vs baseline: 1.3373x; 1.3373x over previous
import functools

import jax
import jax.numpy as jnp
from jax import lax
from jax.experimental import pallas as pl
from jax.experimental.pallas import tpu as pltpu
from jax.experimental.pallas import tpu_sc as plsc

VSX, VSY = 0.1, 0.1
XMIN, YMIN, ZMIN, XMAX, YMAX, ZMAX = 0.0, -25.6, -3.0, 51.2, 25.6, 3.0
GX = 512
GY = 512
SIG2 = (VSX * 0.5) ** 2

NW = 32
N_LID = 2 * 131072
N_RAD = 2 * 16384
BASES = (0, N_LID, N_LID + N_RAD, 2 * N_LID + N_RAD)
NPTS = (N_LID, N_RAD, N_LID, N_RAD)
TOT = 2 * (N_LID + N_RAD)
EXOFF = (0, BASES[1], 0, BASES[3] - N_RAD)
PSTR = 2048
NSTR = (2 * GX * GY) // PSTR
NBIN = 4 * (NSTR + 1)
NBINP = 1056
CH = 1024
BPAD = TOT + CH + 8
NROUNDS = (4 * NSTR) // NW

_mesh = plsc.VectorSubcoreMesh(core_axis_name="c", subcore_axis_name="s")
_cparams = pltpu.CompilerParams(needs_layout_passes=False)


def _splat(ref, idx):
    return plsc.load_gather(ref, [jnp.full((16,), idx, jnp.int32)])


def _sread(ref, idx):
    return jnp.max(plsc.load_gather(ref, [jnp.full((16,), idx, jnp.int32)]))


def _cells(tx, ty, tz):
    u = (tx - XMIN) / jnp.float32(VSX)
    v = (ty - YMIN) / jnp.float32(VSY)
    cxi = u.astype(jnp.int32)
    cx = cxi - (cxi.astype(jnp.float32) > u).astype(jnp.int32)
    cyi = v.astype(jnp.int32)
    cy = cyi - (cyi.astype(jnp.float32) > v).astype(jnp.int32)
    valid = ((cx >= 0) & (cx < GX) & (cy >= 0) & (cy < GY)
             & (tz >= ZMIN) & (tz < ZMAX))
    return cx, cy, valid


def _p1_body(x_hbm, y_hbm, z_hbm, hist_out, xv, yv, zv, hv):
    wid = lax.axis_index("s") * 2 + lax.axis_index("c")

    def zero(i, _):
        hv[pl.ds(i * 16, 16)] = jnp.zeros((16,), jnp.int32)
        return 0
    lax.fori_loop(0, NBINP // 16, zero, 0)

    ones = jnp.ones((16,), jnp.int32)
    for c in range(4):
        per_w = NPTS[c] // NW
        start = BASES[c] + wid * per_w
        bofs = jnp.where(wid >= 16, GX * GY, 0)
        for k in range(per_w // CH):
            off = pl.multiple_of(start + k * CH, 8)
            pltpu.sync_copy(x_hbm.at[pl.ds(off, CH)], xv)
            pltpu.sync_copy(y_hbm.at[pl.ds(off, CH)], yv)
            pltpu.sync_copy(z_hbm.at[pl.ds(off, CH)], zv)

            def body(i, _):
                tx = xv[pl.ds(i * 16, 16)]
                ty = yv[pl.ds(i * 16, 16)]
                tz = zv[pl.ds(i * 16, 16)]
                cx, cy, valid = _cells(tx, ty, tz)
                pid = bofs + cx * GY + cy
                sid = jnp.where(valid, jnp.right_shift(pid, 11),
                                jnp.full((16,), NSTR, jnp.int32))
                plsc.addupdate_scatter(hv, [sid + (c * (NSTR + 1))], ones)
                return 0
            lax.fori_loop(0, CH // 16, body, 0)
    pltpu.sync_copy(hv, hist_out.at[wid])


def _p2_body(x_hbm, y_hbm, z_hbm, r3_hbm, r4_hbm, r5_hbm, dstb_hbm,
             bx, by, bz, bl, br3, br4, br5,
             xv, yv, zv, r3v, r4v, r5v, stx, sty, stz, stl, sr3, sr4, sr5,
             dstv, curs, sem):
    wid = lax.axis_index("s") * 2 + lax.axis_index("c")
    pltpu.sync_copy(dstb_hbm.at[wid], curs)

    for c in range(4):
        per_w = NPTS[c] // NW
        start = BASES[c] + wid * per_w
        bofs = jnp.where(wid >= 16, GX * GY, 0)
        is_rad = c % 2 == 1
        for k in range(per_w // CH):
            off = pl.multiple_of(start + k * CH, 8)
            pltpu.sync_copy(x_hbm.at[pl.ds(off, CH)], xv)
            pltpu.sync_copy(y_hbm.at[pl.ds(off, CH)], yv)
            pltpu.sync_copy(z_hbm.at[pl.ds(off, CH)], zv)
            if is_rad:
                exo = pl.multiple_of(off - EXOFF[c], 8)
                pltpu.sync_copy(r3_hbm.at[pl.ds(exo, CH)], r3v)
                pltpu.sync_copy(r4_hbm.at[pl.ds(exo, CH)], r4v)
                pltpu.sync_copy(r5_hbm.at[pl.ds(exo, CH)], r5v)

            def body(i, _):
                tx = xv[pl.ds(i * 16, 16)]
                ty = yv[pl.ds(i * 16, 16)]
                tz = zv[pl.ds(i * 16, 16)]
                cx, cy, valid = _cells(tx, ty, tz)
                pid = bofs + cx * GY + cy
                sid = jnp.where(valid, jnp.right_shift(pid, 11),
                                jnp.full((16,), NSTR, jnp.int32))
                binv = sid + (c * (NSTR + 1))
                cnt1, last = plsc.scan_count(binv)
                cur = plsc.load_gather(curs, [binv])
                dst = cur + cnt1 - 1
                plsc.store_scatter(curs, [binv], cur + cnt1, mask=last)
                sl = i * 16
                stx[pl.ds(sl, 16)] = tx
                sty[pl.ds(sl, 16)] = ty
                stz[pl.ds(sl, 16)] = tz
                stl[pl.ds(sl, 16)] = pid & jnp.full((16,), PSTR - 1, jnp.int32)
                dstv[pl.ds(sl, 16)] = dst
                if is_rad:
                    sr3[pl.ds(sl, 16)] = r3v[pl.ds(sl, 16)]
                    sr4[pl.ds(sl, 16)] = r4v[pl.ds(sl, 16)]
                    sr5[pl.ds(sl, 16)] = r5v[pl.ds(sl, 16)]
                return 0
            lax.fori_loop(0, CH // 16, body, 0)
            cps = [pltpu.async_copy(stx, bx.at[dstv], sem),
                   pltpu.async_copy(sty, by.at[dstv], sem),
                   pltpu.async_copy(stz, bz.at[dstv], sem),
                   pltpu.async_copy(stl, bl.at[dstv], sem)]
            if is_rad:
                cps += [pltpu.async_copy(sr3, br3.at[dstv], sem),
                        pltpu.async_copy(sr4, br4.at[dstv], sem),
                        pltpu.async_copy(sr5, br5.at[dstv], sem)]
            for cp in cps:
                cp.wait()


def _p3_body(bx_hbm, by_hbm, bz_hbm, bl_hbm, br3_hbm, br4_hbm, br5_hbm,
             bnd_hbm, wt_hbm, out_hbm,
             cntv, sxv, syv, hsv, mxv, bxv, byv, bzv, blv, r3v, r4v, r5v,
             hst, outv, bndv, wtv):
    wid = lax.axis_index("s") * 2 + lax.axis_index("c")
    lane = lax.iota(jnp.int32, 16)
    pltpu.sync_copy(bnd_hbm, bndv)
    pltpu.sync_copy(wt_hbm, wtv)

    def accum_chunk(coff, start, end, n_feat, w_off, w_stride):
        for jc in range(8):
            def hbody(i, _):
                sl = i * 16
                x = bxv[pl.ds(sl, 16)]
                y = byv[pl.ds(sl, 16)]
                z = bzv[pl.ds(sl, 16)]
                if n_feat == 6:
                    r3 = r3v[pl.ds(sl, 16)]
                    r4 = r4v[pl.ds(sl, 16)]
                    r5 = r5v[pl.ds(sl, 16)]
                for jj in range(4):
                    j = jc * 4 + jj
                    base = (w_off + j * w_stride) * 16
                    w = [wtv[pl.ds(base + t * 16, 16)] for t in range(n_feat + 1)]
                    h = x * w[0] + y * w[1] + z * w[2]
                    if n_feat == 6:
                        h = h + r3 * w[3] + r4 * w[4] + r5 * w[5]
                    h = jnp.maximum(h + w[n_feat], jnp.float32(0.0))
                    hst[pl.ds(j * CH + sl, 16)] = h
                return 0
            lax.fori_loop(0, CH // 16, hbody, 0)

        def abody(i, _):
            sl = i * 16
            x = bxv[pl.ds(sl, 16)]
            y = byv[pl.ds(sl, 16)]
            pos = coff + sl + lane
            ptm = (pos >= start) & (pos < end)
            loc = blv[pl.ds(sl, 16)] & jnp.full((16,), PSTR - 1, jnp.int32)
            cnt1, _last = plsc.scan_count(loc, mask=ptm)
            rank = cnt1 - 1
            gmax = jnp.max(jnp.where(ptm, rank, 0)) + 1

            def gbody(g, _):
                grp = ptm & (rank == g)
                cf = plsc.load_gather(cntv, [loc], mask=grp)
                first = (cf == 0.0) & grp
                sxo = plsc.load_gather(sxv, [loc], mask=grp)
                plsc.store_scatter(sxv, [loc],
                                   jnp.where(first, x, sxo + x), mask=grp)
                syo = plsc.load_gather(syv, [loc], mask=grp)
                plsc.store_scatter(syv, [loc],
                                   jnp.where(first, y, syo + y), mask=grp)
                plsc.store_scatter(cntv, [loc],
                                   jnp.where(first, jnp.float32(1.0), cf + 1.0),
                                   mask=grp)
                hso = plsc.load_gather(hsv, [loc], mask=grp)
                dsum = jnp.zeros((16,), jnp.float32)
                for j in range(32):
                    hj = hst[pl.ds(j * CH + sl, 16)]
                    jv = jnp.full((16,), j, jnp.int32)
                    mo = plsc.load_gather(mxv, [jv, loc], mask=grp)
                    moe = jnp.where(first, jnp.float32(0.0), mo)
                    mn = jnp.maximum(moe, hj)
                    dsum = dsum + (mn - moe)
                    plsc.store_scatter(mxv, [jv, loc], mn, mask=grp)
                plsc.store_scatter(hsv, [loc],
                                   jnp.where(first, dsum, hso + dsum), mask=grp)
                return 0
            lax.fori_loop(0, gmax, gbody, 0)
            return 0
        lax.fori_loop(0, CH // 16, abody, 0)

    def round_body(r, _):
        gs = r * NW + wid
        cloud = gs // 256
        stripe = gs & (NSTR - 1)
        bidx = cloud * (NSTR + 1) + stripe
        start = _sread(bndv, bidx)
        end = _sread(bndv, bidx + 1)
        start8 = start & -8
        nch = (end - start8 + (CH - 1)) // CH

        def zero(i, _):
            cntv[pl.ds(i * 16, 16)] = jnp.zeros((16,), jnp.float32)
            return 0
        lax.fori_loop(0, PSTR // 16, zero, 0)

        is_rad = (cloud & 1) == 1

        def chunks(n_feat):
            def cbody(kc, _):
                coff = pl.multiple_of(start8 + kc * CH, 8)
                pltpu.sync_copy(bx_hbm.at[pl.ds(coff, CH)], bxv)
                pltpu.sync_copy(by_hbm.at[pl.ds(coff, CH)], byv)
                pltpu.sync_copy(bz_hbm.at[pl.ds(coff, CH)], bzv)
                pltpu.sync_copy(bl_hbm.at[pl.ds(coff, CH)], blv)
                if n_feat == 6:
                    pltpu.sync_copy(br3_hbm.at[pl.ds(coff, CH)], r3v)
                    pltpu.sync_copy(br4_hbm.at[pl.ds(coff, CH)], r4v)
                    pltpu.sync_copy(br5_hbm.at[pl.ds(coff, CH)], r5v)
                if n_feat == 6:
                    accum_chunk(coff, start, end, 6, 128, 8)
                else:
                    accum_chunk(coff, start, end, 3, 0, 4)
                return 0
            lax.fori_loop(0, nch, cbody, 0)

        @pl.when(jnp.logical_not(is_rad))
        def _():
            chunks(3)

        @pl.when(is_rad)
        def _():
            chunks(6)

        k1 = jnp.float32(-1.0 / (2.0 * SIG2))

        def sbody(i, _):
            sl = i * 16
            c = cntv[pl.ds(sl, 16)]
            occ = c > 0.0
            den = jnp.maximum(c, jnp.float32(1.0))
            mx = sxv[pl.ds(sl, 16)] / den
            my = syv[pl.ds(sl, 16)] / den
            pidv = stripe * PSTR + sl + lane
            rem = pidv & jnp.full((16,), GX * GY - 1, jnp.int32)
            gx = jnp.right_shift(rem, 9)
            gy = rem & jnp.full((16,), GY - 1, jnp.int32)
            xc = XMIN + (gx.astype(jnp.float32) + 0.5) * VSX
            yc = YMIN + (gy.astype(jnp.float32) + 0.5) * VSY
            d2 = (mx - xc) * (mx - xc) + (my - yc) * (my - yc)
            gsn = jnp.exp(d2 * k1)
            p = hsv[pl.ds(sl, 16)] * jnp.float32(1.0 / 32.0)
            e = jnp.exp(p * 2.0)
            th = 1.0 - 2.0 / (e + 1.0)
            outv[pl.ds(sl, 16)] = jnp.where(occ, gsn * th, jnp.float32(0.0))
            return 0
        lax.fori_loop(0, PSTR // 16, sbody, 0)
        pltpu.sync_copy(outv, out_hbm.at[pl.ds(pl.multiple_of(gs * PSTR, 8), PSTR)])
        return 0
    lax.fori_loop(0, NROUNDS, round_body, 0)


_phase1 = functools.partial(
    pl.kernel, _p1_body, mesh=_mesh, compiler_params=_cparams,
    out_type=jax.ShapeDtypeStruct((NW, NBINP), jnp.int32),
    scratch_types=[pltpu.VMEM((CH,), jnp.float32),
                   pltpu.VMEM((CH,), jnp.float32),
                   pltpu.VMEM((CH,), jnp.float32),
                   pltpu.VMEM((NBINP,), jnp.int32)])

_phase2 = functools.partial(
    pl.kernel, _p2_body, mesh=_mesh, compiler_params=_cparams,
    out_type=(jax.ShapeDtypeStruct((BPAD,), jnp.float32),
              jax.ShapeDtypeStruct((BPAD,), jnp.float32),
              jax.ShapeDtypeStruct((BPAD,), jnp.float32),
              jax.ShapeDtypeStruct((BPAD,), jnp.int32),
              jax.ShapeDtypeStruct((BPAD,), jnp.float32),
              jax.ShapeDtypeStruct((BPAD,), jnp.float32),
              jax.ShapeDtypeStruct((BPAD,), jnp.float32)),
    scratch_types=[pltpu.VMEM((CH,), jnp.float32)] * 6
                  + [pltpu.VMEM((CH,), jnp.float32)] * 3
                  + [pltpu.VMEM((CH,), jnp.int32)]
                  + [pltpu.VMEM((CH,), jnp.float32)] * 3
                  + [pltpu.VMEM((CH,), jnp.int32),
                     pltpu.VMEM((NBINP,), jnp.int32),
                     pltpu.SemaphoreType.DMA])

_phase3 = functools.partial(
    pl.kernel, _p3_body, mesh=_mesh, compiler_params=_cparams,
    out_type=jax.ShapeDtypeStruct((4 * 2 * GX * GY,), jnp.float32),
    scratch_types=[pltpu.VMEM((PSTR,), jnp.float32),
                   pltpu.VMEM((PSTR,), jnp.float32),
                   pltpu.VMEM((PSTR,), jnp.float32),
                   pltpu.VMEM((PSTR,), jnp.float32),
                   pltpu.VMEM((32, PSTR), jnp.float32),
                   pltpu.VMEM((CH,), jnp.float32),
                   pltpu.VMEM((CH,), jnp.float32),
                   pltpu.VMEM((CH,), jnp.float32),
                   pltpu.VMEM((CH,), jnp.int32),
                   pltpu.VMEM((CH,), jnp.float32),
                   pltpu.VMEM((CH,), jnp.float32),
                   pltpu.VMEM((CH,), jnp.float32),
                   pltpu.VMEM((32 * CH,), jnp.float32),
                   pltpu.VMEM((PSTR,), jnp.float32),
                   pltpu.VMEM((NBINP + 8,), jnp.int32),
                   pltpu.VMEM((384 * 16,), jnp.float32)])


def kernel(pc0, pc1, radar_pc0, radar_pc1, pose0, pose1,
           W_lidar, b_lidar, W_radar, b_radar):
    B = pc0.shape[0]
    pose_0to1 = jnp.linalg.inv(pose1) @ pose0
    Rm = pose_0to1[:, :3, :3]
    t = pose_0to1[:, :3, 3]
    tpc0 = jnp.einsum("bnc,bdc->bnd", pc0, Rm) + t[:, None, :]
    rxyz = jnp.einsum("bnc,bdc->bnd", radar_pc0[..., :3], Rm) + t[:, None, :]

    X = jnp.concatenate([tpc0[..., 0].reshape(-1), rxyz[..., 0].reshape(-1),
                         pc1[..., 0].reshape(-1), radar_pc1[..., 0].reshape(-1)])
    Y = jnp.concatenate([tpc0[..., 1].reshape(-1), rxyz[..., 1].reshape(-1),
                         pc1[..., 1].reshape(-1), radar_pc1[..., 1].reshape(-1)])
    Z = jnp.concatenate([tpc0[..., 2].reshape(-1), rxyz[..., 2].reshape(-1),
                         pc1[..., 2].reshape(-1), radar_pc1[..., 2].reshape(-1)])
    R3 = jnp.concatenate([radar_pc0[..., 3].reshape(-1),
                          radar_pc1[..., 3].reshape(-1)])
    R4 = jnp.concatenate([radar_pc0[..., 4].reshape(-1),
                          radar_pc1[..., 4].reshape(-1)])
    R5 = jnp.concatenate([radar_pc0[..., 5].reshape(-1),
                          radar_pc1[..., 5].reshape(-1)])

    hist = _phase1()(X, Y, Z)

    h = hist[:, :NBIN].astype(jnp.int32)
    flat = h.T.reshape(-1)
    cum = jnp.concatenate([jnp.zeros((1,), jnp.int32),
                           jnp.cumsum(flat)[:-1].astype(jnp.int32)])
    dstb = cum.reshape(NBIN, NW).T
    dstb = jnp.concatenate(
        [dstb, jnp.zeros((NW, NBINP - NBIN), jnp.int32)], axis=1)
    bounds = jnp.concatenate([cum[::NW],
                              jnp.full((1,), TOT, jnp.int32),
                              jnp.zeros((NBINP + 8 - NBIN - 1,), jnp.int32)])

    bx, by, bz, bl, br3, br4, br5 = _phase2()(X, Y, Z, R3, R4, R5, dstb)

    wl = jnp.concatenate([W_lidar.T, b_lidar[:, None]], axis=1).reshape(-1)
    wr = jnp.concatenate([W_radar.T, b_radar[:, None],
                          jnp.zeros((32, 1), jnp.float32)], axis=1).reshape(-1)
    wt = jnp.repeat(jnp.concatenate([wl, wr]), 16)

    out = _phase3()(bx, by, bz, bl, br3, br4, br5, bounds, wt)
    return out.reshape(4, B, GX, GY)
